# Initial kernel scaffold; baseline (speedup 1.0000x reference)
import jax
import jax.numpy as jnp
from jax.experimental import pallas as pl


def kernel(q, k, v):
    raise NotImplementedError("write your pallas kernel here")


# speedup vs baseline: 8.3448x; 8.3448x over previous
import functools
import math

import jax
import jax.numpy as jnp
import numpy as np
from jax.experimental import pallas as pl
from jax.experimental.pallas import tpu as pltpu


def _dft_constants(L: int, LFP: int):
    Lf = L // 2 + 1
    f = np.arange(LFP, dtype=np.int64)[:, None]
    t = np.arange(L, dtype=np.int64)[None, :]
    ang = 2.0 * np.pi * ((f * t) % L).astype(np.float64) / L
    ct = np.cos(ang)
    st = np.sin(ang)
    ct[Lf:, :] = 0.0
    st[Lf:, :] = 0.0
    return jnp.asarray(ct, jnp.float32), jnp.asarray(st, jnp.float32)


_FB = 128


def _corr_kernel(L, Lf, q_ref, k_ref, ct_ref, st_ref, corr_ref):
    q = q_ref[0]
    k = k_ref[0]
    dh = q.shape[-1]
    lfp = ct_ref.shape[0]
    x = jnp.concatenate([q, k], axis=1)
    dn = (((1,), (0,)), ((), ()))
    dn2 = (((0,), (0,)), ((), ()))
    hi = jax.lax.Precision.HIGHEST

    def body(i, acc):
        f0 = i * _FB
        ct = ct_ref[pl.ds(f0, _FB), :]
        st = st_ref[pl.ds(f0, _FB), :]
        a = jax.lax.dot_general(ct, x, dn, precision=hi,
                                preferred_element_type=jnp.float32)
        b = jax.lax.dot_general(st, x, dn, precision=hi,
                                preferred_element_type=jnp.float32)
        aq, ak = a[:, :dh], a[:, dh:]
        bq, bk = b[:, :dh], b[:, dh:]
        fidx = f0 + jax.lax.broadcasted_iota(jnp.int32, (_FB, 1), 0)
        w = jnp.where((fidx == 0) | (fidx == L // 2), 1.0, 2.0) / (L * dh)
        aq = aq * w
        bq = bq * w
        res_re = jnp.sum(aq * ak + bq * bk, axis=1, keepdims=True)
        res_im = jnp.sum(aq * bk - bq * ak, axis=1, keepdims=True)
        c_re = jax.lax.dot_general(res_re, ct, dn2, precision=hi,
                                   preferred_element_type=jnp.float32)
        c_im = jax.lax.dot_general(res_im, st, dn2, precision=hi,
                                   preferred_element_type=jnp.float32)
        return acc + c_re - c_im

    corr = jax.lax.fori_loop(0, lfp // _FB, body,
                             jnp.zeros((1, L), jnp.float32))
    corr_ref[0] = corr


def _agg_kernel(K, corr_ref, v_ref, out_ref):
    r = corr_ref[0]
    L = r.shape[-1]
    iota = jax.lax.broadcasted_iota(jnp.int32, r.shape, 1)
    neg = jnp.float32(-jnp.inf)
    vals = []
    idxs = []
    for _ in range(K):
        m = jnp.max(r)
        i = jnp.min(jnp.where(r == m, iota, L))
        vals.append(m)
        idxs.append(i)
        r = jnp.where(iota == i, neg, r)
    m0 = functools.reduce(jnp.maximum, vals)
    es = [jnp.exp(w - m0) for w in vals]
    s = functools.reduce(lambda x, y: x + y, es)
    v = v_ref[0]
    acc = (es[0] / s) * pltpu.roll(v, idxs[0], 0)
    for j in range(1, K):
        acc = acc + (es[j] / s) * pltpu.roll(v, idxs[j], 0)
    out_ref[0] = acc


def kernel(q, k, v):
    B, H, L, Dh = q.shape
    BH = B * H
    Lf = L // 2 + 1
    LFP = ((Lf + 127) // 128) * 128
    K = max(1, int(math.log(L + 1)))

    ct, st = _dft_constants(L, LFP)
    q3 = q.reshape(BH, L, Dh)
    k3 = k.reshape(BH, L, Dh)
    v3 = v.reshape(BH, L, Dh)

    corr = pl.pallas_call(
        functools.partial(_corr_kernel, L, Lf),
        grid=(BH,),
        in_specs=[
            pl.BlockSpec((1, L, Dh), lambda i: (i, 0, 0)),
            pl.BlockSpec((1, L, Dh), lambda i: (i, 0, 0)),
            pl.BlockSpec((LFP, L), lambda i: (0, 0)),
            pl.BlockSpec((LFP, L), lambda i: (0, 0)),
        ],
        out_specs=pl.BlockSpec((1, 1, L), lambda i: (i, 0, 0)),
        out_shape=jax.ShapeDtypeStruct((BH, 1, L), jnp.float32),
        compiler_params=pltpu.CompilerParams(
            dimension_semantics=("arbitrary",)),
    )(q3, k3, ct, st)

    out = pl.pallas_call(
        functools.partial(_agg_kernel, K),
        grid=(BH,),
        in_specs=[
            pl.BlockSpec((1, 1, L), lambda i: (i, 0, 0)),
            pl.BlockSpec((1, L, Dh), lambda i: (i, 0, 0)),
        ],
        out_specs=pl.BlockSpec((1, L, Dh), lambda i: (i, 0, 0)),
        out_shape=jax.ShapeDtypeStruct((BH, L, Dh), jnp.float32),
        compiler_params=pltpu.CompilerParams(
            dimension_semantics=("arbitrary",)),
    )(corr, v3)

    return out.reshape(B, H, L, Dh)


# speedup vs baseline: 9.2926x; 1.1136x over previous
import functools
import math

import jax
import jax.numpy as jnp
import numpy as np
from jax.experimental import pallas as pl
from jax.experimental.pallas import tpu as pltpu


def _dft_constants(L: int, LFP: int):
    Lf = L // 2 + 1
    f = np.arange(LFP, dtype=np.int64)[:, None]
    t = np.arange(L, dtype=np.int64)[None, :]
    ang = 2.0 * np.pi * ((f * t) % L).astype(np.float64) / L
    ct = np.cos(ang)
    st = np.sin(ang)
    ct[Lf:, :] = 0.0
    st[Lf:, :] = 0.0
    return jnp.asarray(ct, jnp.float32), jnp.asarray(st, jnp.float32)


_FB = 128


def _corr_kernel(L, Lf, q_ref, k_ref, ct_ref, st_ref, corr_ref):
    q = q_ref[0]
    k = k_ref[0]
    dh = q.shape[-1]
    lfp = ct_ref.shape[0]
    x = jnp.concatenate([q, k], axis=1)
    dn = (((1,), (0,)), ((), ()))
    dn2 = (((0,), (0,)), ((), ()))
    hi = jax.lax.Precision.HIGHEST

    def body(i, acc):
        f0 = i * _FB
        ct = ct_ref[pl.ds(f0, _FB), :]
        st = st_ref[pl.ds(f0, _FB), :]
        a = jax.lax.dot_general(ct, x, dn, precision=hi,
                                preferred_element_type=jnp.float32)
        b = jax.lax.dot_general(st, x, dn, precision=hi,
                                preferred_element_type=jnp.float32)
        aq, ak = a[:, :dh], a[:, dh:]
        bq, bk = b[:, :dh], b[:, dh:]
        fidx = f0 + jax.lax.broadcasted_iota(jnp.int32, (_FB, 1), 0)
        w = jnp.where((fidx == 0) | (fidx == L // 2), 1.0, 2.0) / (L * dh)
        aq = aq * w
        bq = bq * w
        res_re = jnp.sum(aq * ak + bq * bk, axis=1, keepdims=True)
        res_im = jnp.sum(aq * bk - bq * ak, axis=1, keepdims=True)
        c_re = jax.lax.dot_general(res_re, ct, dn2, precision=hi,
                                   preferred_element_type=jnp.float32)
        c_im = jax.lax.dot_general(res_im, st, dn2, precision=hi,
                                   preferred_element_type=jnp.float32)
        return acc + c_re - c_im

    corr = jax.lax.fori_loop(0, lfp // _FB, body,
                             jnp.zeros((1, L), jnp.float32))
    corr_ref[0] = corr


def _agg_kernel(K, corr_ref, v_ref, out_ref, v2_ref):
    r = corr_ref[0]
    L = r.shape[-1]
    iota = jax.lax.broadcasted_iota(jnp.int32, r.shape, 1)
    neg = jnp.float32(-jnp.inf)
    vals = []
    idxs = []
    for _ in range(K):
        m = jnp.max(r)
        i = jnp.min(jnp.where(r == m, iota, L))
        vals.append(m)
        idxs.append(i)
        r = jnp.where(iota == i, neg, r)
    m0 = functools.reduce(jnp.maximum, vals)
    es = [jnp.exp(w - m0) for w in vals]
    s = functools.reduce(lambda x, y: x + y, es)
    v = v_ref[0]
    v2_ref[0:L, :] = v
    v2_ref[L:2 * L, :] = v
    acc = (es[0] / s) * v2_ref[pl.ds(L - idxs[0], L), :]
    for j in range(1, K):
        acc = acc + (es[j] / s) * v2_ref[pl.ds(L - idxs[j], L), :]
    out_ref[0] = acc


def kernel(q, k, v):
    B, H, L, Dh = q.shape
    BH = B * H
    Lf = L // 2 + 1
    LFP = ((Lf + 127) // 128) * 128
    K = max(1, int(math.log(L + 1)))

    ct, st = _dft_constants(L, LFP)
    q3 = q.reshape(BH, L, Dh)
    k3 = k.reshape(BH, L, Dh)
    v3 = v.reshape(BH, L, Dh)

    corr = pl.pallas_call(
        functools.partial(_corr_kernel, L, Lf),
        grid=(BH,),
        in_specs=[
            pl.BlockSpec((1, L, Dh), lambda i: (i, 0, 0)),
            pl.BlockSpec((1, L, Dh), lambda i: (i, 0, 0)),
            pl.BlockSpec((LFP, L), lambda i: (0, 0)),
            pl.BlockSpec((LFP, L), lambda i: (0, 0)),
        ],
        out_specs=pl.BlockSpec((1, 1, L), lambda i: (i, 0, 0)),
        out_shape=jax.ShapeDtypeStruct((BH, 1, L), jnp.float32),
        compiler_params=pltpu.CompilerParams(
            dimension_semantics=("arbitrary",)),
    )(q3, k3, ct, st)

    out = pl.pallas_call(
        functools.partial(_agg_kernel, K),
        grid=(BH,),
        in_specs=[
            pl.BlockSpec((1, 1, L), lambda i: (i, 0, 0)),
            pl.BlockSpec((1, L, Dh), lambda i: (i, 0, 0)),
        ],
        out_specs=pl.BlockSpec((1, L, Dh), lambda i: (i, 0, 0)),
        out_shape=jax.ShapeDtypeStruct((BH, L, Dh), jnp.float32),
        scratch_shapes=[pltpu.VMEM((2 * L, Dh), jnp.float32)],
        compiler_params=pltpu.CompilerParams(
            dimension_semantics=("arbitrary",)),
    )(corr, v3)

    return out.reshape(B, H, L, Dh)


# speedup vs baseline: 13.9710x; 1.5035x over previous
import functools
import math

import jax
import jax.numpy as jnp
import numpy as np
from jax.experimental import pallas as pl
from jax.experimental.pallas import tpu as pltpu


def _dft_constants(L: int, LFP: int):
    Lf = L // 2 + 1
    f = np.arange(LFP, dtype=np.int64)[:, None]
    t = np.arange(L, dtype=np.int64)[None, :]
    ang = 2.0 * np.pi * ((f * t) % L).astype(np.float64) / L
    out = []
    for m in (np.cos(ang), np.sin(ang)):
        m[Lf:, :] = 0.0
        m32 = m.astype(np.float32)
        hi = m32.astype(jnp.bfloat16)
        lo = (m32 - hi.astype(np.float32)).astype(jnp.bfloat16)
        out.append((jnp.asarray(hi), jnp.asarray(lo)))
    return out[0], out[1]


def _dot3(ah, al, bh, bl, dn):
    kw = dict(dimension_numbers=dn, preferred_element_type=jnp.float32)
    return (jax.lax.dot_general(ah, bh, **kw)
            + jax.lax.dot_general(ah, bl, **kw)
            + jax.lax.dot_general(al, bh, **kw))


def _split_bf16(x):
    hi = x.astype(jnp.bfloat16)
    lo = (x - hi.astype(jnp.float32)).astype(jnp.bfloat16)
    return hi, lo


_FB = 128


def _corr_kernel(L, Lf, q_ref, k_ref, cth_ref, ctl_ref, sth_ref, stl_ref,
                 corr_ref):
    q = q_ref[0]
    k = k_ref[0]
    dh = q.shape[-1]
    lfp = cth_ref.shape[0]
    x = jnp.concatenate([q, k], axis=1)
    xh, xl = _split_bf16(x)
    dn = (((1,), (0,)), ((), ()))
    dn2 = (((0,), (0,)), ((), ()))

    def body(i, acc):
        f0 = i * _FB
        cth = cth_ref[pl.ds(f0, _FB), :]
        ctl = ctl_ref[pl.ds(f0, _FB), :]
        sth = sth_ref[pl.ds(f0, _FB), :]
        stl = stl_ref[pl.ds(f0, _FB), :]
        a = _dot3(cth, ctl, xh, xl, dn)
        b = _dot3(sth, stl, xh, xl, dn)
        aq, ak = a[:, :dh], a[:, dh:]
        bq, bk = b[:, :dh], b[:, dh:]
        fidx = f0 + jax.lax.broadcasted_iota(jnp.int32, (_FB, 1), 0)
        w = jnp.where((fidx == 0) | (fidx == L // 2), 1.0, 2.0) / (L * dh)
        aq = aq * w
        bq = bq * w
        res_re = jnp.sum(aq * ak + bq * bk, axis=1, keepdims=True)
        res_im = jnp.sum(aq * bk - bq * ak, axis=1, keepdims=True)
        reh, rel = _split_bf16(res_re)
        imh, iml = _split_bf16(res_im)
        c_re = _dot3(reh, rel, cth, ctl, dn2)
        c_im = _dot3(imh, iml, sth, stl, dn2)
        return acc + c_re - c_im

    corr = jax.lax.fori_loop(0, lfp // _FB, body,
                             jnp.zeros((1, L), jnp.float32))
    corr_ref[0] = corr


def _agg_kernel(K, corr_ref, v_ref, out_ref, v2_ref):
    r = corr_ref[0]
    L = r.shape[-1]
    iota = jax.lax.broadcasted_iota(jnp.int32, r.shape, 1)
    neg = jnp.float32(-jnp.inf)
    vals = []
    idxs = []
    for _ in range(K):
        m = jnp.max(r)
        i = jnp.min(jnp.where(r == m, iota, L))
        vals.append(m)
        idxs.append(i)
        r = jnp.where(iota == i, neg, r)
    m0 = functools.reduce(jnp.maximum, vals)
    es = [jnp.exp(w - m0) for w in vals]
    s = functools.reduce(lambda x, y: x + y, es)
    v = v_ref[0]
    v2_ref[0:L, :] = v
    v2_ref[L:2 * L, :] = v
    acc = (es[0] / s) * v2_ref[pl.ds(L - idxs[0], L), :]
    for j in range(1, K):
        acc = acc + (es[j] / s) * v2_ref[pl.ds(L - idxs[j], L), :]
    out_ref[0] = acc


def kernel(q, k, v):
    B, H, L, Dh = q.shape
    BH = B * H
    Lf = L // 2 + 1
    LFP = ((Lf + 127) // 128) * 128
    K = max(1, int(math.log(L + 1)))

    (cth, ctl), (sth, stl) = _dft_constants(L, LFP)
    q3 = q.reshape(BH, L, Dh)
    k3 = k.reshape(BH, L, Dh)
    v3 = v.reshape(BH, L, Dh)

    corr = pl.pallas_call(
        functools.partial(_corr_kernel, L, Lf),
        grid=(BH,),
        in_specs=[
            pl.BlockSpec((1, L, Dh), lambda i: (i, 0, 0)),
            pl.BlockSpec((1, L, Dh), lambda i: (i, 0, 0)),
            pl.BlockSpec((LFP, L), lambda i: (0, 0)),
            pl.BlockSpec((LFP, L), lambda i: (0, 0)),
            pl.BlockSpec((LFP, L), lambda i: (0, 0)),
            pl.BlockSpec((LFP, L), lambda i: (0, 0)),
        ],
        out_specs=pl.BlockSpec((1, 1, L), lambda i: (i, 0, 0)),
        out_shape=jax.ShapeDtypeStruct((BH, 1, L), jnp.float32),
        compiler_params=pltpu.CompilerParams(
            dimension_semantics=("arbitrary",)),
    )(q3, k3, cth, ctl, sth, stl)

    out = pl.pallas_call(
        functools.partial(_agg_kernel, K),
        grid=(BH,),
        in_specs=[
            pl.BlockSpec((1, 1, L), lambda i: (i, 0, 0)),
            pl.BlockSpec((1, L, Dh), lambda i: (i, 0, 0)),
        ],
        out_specs=pl.BlockSpec((1, L, Dh), lambda i: (i, 0, 0)),
        out_shape=jax.ShapeDtypeStruct((BH, L, Dh), jnp.float32),
        scratch_shapes=[pltpu.VMEM((2 * L, Dh), jnp.float32)],
        compiler_params=pltpu.CompilerParams(
            dimension_semantics=("arbitrary",)),
    )(corr, v3)

    return out.reshape(B, H, L, Dh)


# speedup vs baseline: 24.6613x; 1.7652x over previous
import functools
import math

import jax
import jax.numpy as jnp
import numpy as np
from jax.experimental import pallas as pl
from jax.experimental.pallas import tpu as pltpu


def _dft_constants(L: int, LFP: int):
    Lf = L // 2 + 1
    f = np.arange(LFP, dtype=np.int64)[:, None]
    t = np.arange(L, dtype=np.int64)[None, :]
    ang = 2.0 * np.pi * ((f * t) % L).astype(np.float64) / L
    out = []
    for m in (np.cos(ang), np.sin(ang)):
        m[Lf:, :] = 0.0
        m32 = m.astype(np.float32)
        hi = m32.astype(jnp.bfloat16)
        lo = (m32 - hi.astype(np.float32)).astype(jnp.bfloat16)
        out.append((jnp.asarray(hi), jnp.asarray(lo)))
    return out[0], out[1]


def _dot3(ah, al, bh, bl, dn):
    kw = dict(dimension_numbers=dn, preferred_element_type=jnp.float32)
    return (jax.lax.dot_general(ah, bh, **kw)
            + jax.lax.dot_general(ah, bl, **kw)
            + jax.lax.dot_general(al, bh, **kw))


def _split_bf16(x):
    hi = x.astype(jnp.bfloat16)
    lo = (x - hi.astype(jnp.float32)).astype(jnp.bfloat16)
    return hi, lo


_FB = 256


def _corr_kernel(L, Lf, q_ref, k_ref, cth_ref, ctl_ref, sth_ref, stl_ref,
                 corr_ref):
    dh = q_ref.shape[-1]
    lfp = cth_ref.shape[0]
    x = jnp.concatenate(
        [q_ref[0], k_ref[0], q_ref[1], k_ref[1]], axis=1)
    xh, xl = _split_bf16(x)
    dn = (((1,), (0,)), ((), ()))
    dn2 = (((0,), (0,)), ((), ()))
    nfb = lfp // _FB

    res_re_l = []
    res_im_l = []
    for i in range(nfb):
        f0 = i * _FB
        cth = cth_ref[f0:f0 + _FB, :]
        ctl = ctl_ref[f0:f0 + _FB, :]
        sth = sth_ref[f0:f0 + _FB, :]
        stl = stl_ref[f0:f0 + _FB, :]
        a = _dot3(cth, ctl, xh, xl, dn)
        b = _dot3(sth, stl, xh, xl, dn)
        fidx = f0 + jax.lax.broadcasted_iota(jnp.int32, (_FB, 1), 0)
        w = jnp.where((fidx == 0) | (fidx == L // 2), 1.0, 2.0) / (L * dh)
        res_re = []
        res_im = []
        for h in range(2):
            aq, ak = a[:, 2 * h * dh:(2 * h + 1) * dh] * w, \
                a[:, (2 * h + 1) * dh:(2 * h + 2) * dh]
            bq, bk = b[:, 2 * h * dh:(2 * h + 1) * dh] * w, \
                b[:, (2 * h + 1) * dh:(2 * h + 2) * dh]
            res_re.append(jnp.sum(aq * ak + bq * bk, axis=1, keepdims=True))
            res_im.append(jnp.sum(aq * bk - bq * ak, axis=1, keepdims=True))
        res_re_l.append(jnp.concatenate(res_re, axis=1))
        res_im_l.append(jnp.concatenate(res_im, axis=1))

    res_re_all = jnp.concatenate(res_re_l, axis=0)
    res_im_all = jnp.concatenate(res_im_l, axis=0)
    reh, rel = _split_bf16(res_re_all)
    imh, iml = _split_bf16(res_im_all)

    acc = jnp.zeros((2, L), jnp.float32)
    for i in range(nfb):
        f0 = i * _FB
        sl = (slice(f0, f0 + _FB), slice(None))
        c_re = _dot3(reh[sl], rel[sl], cth_ref[sl], ctl_ref[sl], dn2)
        c_im = _dot3(imh[sl], iml[sl], sth_ref[sl], stl_ref[sl], dn2)
        acc = acc + c_re - c_im
    corr_ref[0] = acc


def _agg_kernel(K, corr_ref, v_ref, out_ref, v2_ref):
    r = corr_ref[0]
    L = r.shape[-1]
    iota = jax.lax.broadcasted_iota(jnp.int32, r.shape, 1)
    neg = jnp.float32(-jnp.inf)
    vals = []
    idxs = []
    for _ in range(K):
        m = jnp.max(r)
        i = jnp.min(jnp.where(r == m, iota, L))
        vals.append(m)
        idxs.append(i)
        r = jnp.where(iota == i, neg, r)
    m0 = functools.reduce(jnp.maximum, vals)
    es = [jnp.exp(w - m0) for w in vals]
    s = functools.reduce(lambda x, y: x + y, es)
    v = v_ref[0]
    v2_ref[0:L, :] = v
    v2_ref[L:2 * L, :] = v
    acc = (es[0] / s) * v2_ref[pl.ds(L - idxs[0], L), :]
    for j in range(1, K):
        acc = acc + (es[j] / s) * v2_ref[pl.ds(L - idxs[j], L), :]
    out_ref[0] = acc


def kernel(q, k, v):
    B, H, L, Dh = q.shape
    BH = B * H
    Lf = L // 2 + 1
    LFP = ((Lf + _FB - 1) // _FB) * _FB
    K = max(1, int(math.log(L + 1)))

    (cth, ctl), (sth, stl) = _dft_constants(L, LFP)
    q3 = q.reshape(BH, L, Dh)
    k3 = k.reshape(BH, L, Dh)
    v3 = v.reshape(BH, L, Dh)

    corr = pl.pallas_call(
        functools.partial(_corr_kernel, L, Lf),
        grid=(BH // 2,),
        in_specs=[
            pl.BlockSpec((2, L, Dh), lambda i: (i, 0, 0)),
            pl.BlockSpec((2, L, Dh), lambda i: (i, 0, 0)),
            pl.BlockSpec((LFP, L), lambda i: (0, 0)),
            pl.BlockSpec((LFP, L), lambda i: (0, 0)),
            pl.BlockSpec((LFP, L), lambda i: (0, 0)),
            pl.BlockSpec((LFP, L), lambda i: (0, 0)),
        ],
        out_specs=pl.BlockSpec((1, 2, L), lambda i: (i, 0, 0)),
        out_shape=jax.ShapeDtypeStruct((BH // 2, 2, L), jnp.float32),
        compiler_params=pltpu.CompilerParams(
            dimension_semantics=("arbitrary",)),
    )(q3, k3, cth, ctl, sth, stl)
    corr = corr.reshape(BH, 1, L)

    out = pl.pallas_call(
        functools.partial(_agg_kernel, K),
        grid=(BH,),
        in_specs=[
            pl.BlockSpec((1, 1, L), lambda i: (i, 0, 0)),
            pl.BlockSpec((1, L, Dh), lambda i: (i, 0, 0)),
        ],
        out_specs=pl.BlockSpec((1, L, Dh), lambda i: (i, 0, 0)),
        out_shape=jax.ShapeDtypeStruct((BH, L, Dh), jnp.float32),
        scratch_shapes=[pltpu.VMEM((2 * L, Dh), jnp.float32)],
        compiler_params=pltpu.CompilerParams(
            dimension_semantics=("arbitrary",)),
    )(corr, v3)

    return out.reshape(B, H, L, Dh)
